# baseline (speedup 1.0000x reference)
import functools

import jax
import jax.numpy as jnp
from jax import lax
from jax.experimental import pallas as pl
from jax.experimental.pallas import tpu as pltpu
from jax.experimental.pallas import tpu_sc as plsc

_BATCH = 16384
_DEPTH = 64
_HALF = 32
_MARGIN = 1.0

_NC = 2
_NS = 16
_NW = _NC * _NS
_CHUNK = 128
_NBUF = 4

_W = 12288


def _pack_body(x0, x1, x2, x3, eel, eeh, eol, eoh, out):
    def cvt(x):
        return x[...].astype(jnp.bfloat16)

    c0, c1, c2, c3 = cvt(x0), cvt(x1), cvt(x2), cvt(x3)

    def dots(a, b, el, eh):
        z = jax.lax.dot_general(a, el[...], (((0,), (0,)), ((), ())),
                                preferred_element_type=jnp.float32)
        z += jax.lax.dot_general(b, eh[...], (((0,), (0,)), ((), ())),
                                 preferred_element_type=jnp.float32)
        return jax.lax.bitcast_convert_type(z, jnp.uint32)

    def pack(a, b):
        ue = dots(a, b, eel, eeh)
        uo = dots(a, b, eol, eoh)
        return jax.lax.bitcast_convert_type(
            (ue >> 16) | (uo & jnp.uint32(0xFFFF0000)), jnp.float32)

    out[...] = jnp.concatenate([pack(c0, c1), pack(c2, c3)], axis=1)


def _tc_pack_table(table, eel, eeh, eol, eoh):
    at = table.T
    n = at.shape[1]
    nin = -(-n // _W)
    nsb = -(-n // (4 * _W))

    def in_spec(k):
        return pl.BlockSpec(
            (_DEPTH, _W), lambda i, k=k: (0, jnp.minimum(4 * i + k, nin - 1)))

    def e_spec():
        return pl.BlockSpec((_DEPTH, _DEPTH), lambda i: (0, 0))

    return pl.pallas_call(
        _pack_body,
        grid=(nsb,),
        in_specs=[in_spec(0), in_spec(1), in_spec(2), in_spec(3),
                  e_spec(), e_spec(), e_spec(), e_spec()],
        out_specs=pl.BlockSpec((_W, 2 * _DEPTH), lambda i: (i, 0)),
        out_shape=jax.ShapeDtypeStruct((nsb * _W, 2 * _DEPTH), jnp.float32),
        compiler_params=pltpu.CompilerParams(
            dimension_semantics=("arbitrary",)),
    )(at, at, at, at, eel, eeh, eol, eoh)


def _sc_gather(pairs, idx):
    n = idx.shape[0]
    per_w = n // _NW
    groups = per_w // (_CHUNK * _NBUF)
    mesh = plsc.VectorSubcoreMesh(core_axis_name="c", subcore_axis_name="s")

    @functools.partial(
        pl.kernel,
        mesh=mesh,
        out_type=jax.ShapeDtypeStruct((n, 128), jnp.float32),
        scratch_types=[
            [pltpu.VMEM((_CHUNK,), jnp.int32) for _ in range(_NBUF)],
            [pltpu.VMEM((_CHUNK, 128), jnp.float32) for _ in range(_NBUF)],
            [pltpu.SemaphoreType.DMA for _ in range(_NBUF)],
        ],
    )
    def gather_k(tab_hbm, idx_hbm, out_hbm, idx_vs, rows_vs, sems):
        wid = lax.axis_index("s") * _NC + lax.axis_index("c")

        def body(g, carry):
            base = wid * per_w + g * (_CHUNK * _NBUF)
            copies = []
            for b in range(_NBUF):
                off = base + b * _CHUNK
                pltpu.sync_copy(idx_hbm.at[pl.ds(off, _CHUNK)], idx_vs[b])
                copies.append(
                    pltpu.async_copy(tab_hbm.at[idx_vs[b]], rows_vs[b], sems[b]))
            for b in range(_NBUF):
                off = base + b * _CHUNK
                copies[b].wait()
                pltpu.sync_copy(rows_vs[b], out_hbm.at[pl.ds(off, _CHUNK)])
            return carry

        lax.fori_loop(0, groups, body, 0)

    return gather_k(pairs, idx)


def _unpack(rows, q):
    w = (rows[:, 0:_HALF] * (q == 0.0).astype(jnp.float32)
         + rows[:, _HALF:2 * _HALF] * (q == 1.0).astype(jnp.float32)
         + rows[:, 2 * _HALF:3 * _HALF] * (q == 2.0).astype(jnp.float32)
         + rows[:, 3 * _HALF:4 * _HALF] * (q == 3.0).astype(jnp.float32))
    u = jax.lax.bitcast_convert_type(w, jnp.uint32)
    ev = jax.lax.bitcast_convert_type(u << 16, jnp.float32)
    od = jax.lax.bitcast_convert_type(u & jnp.uint32(0xFFFF0000), jnp.float32)
    return ev, od


def _mlp_loss_body(ph, pr, pt, nh, nr, nt, qph, qpr, qpt, qnh, qnr, qnt,
                   w1ae, w1ao, w1be, w1bo, b1, w2, b2, w3e, w3o, b3e, b3o,
                   out):
    i = pl.program_id(0)

    @pl.when(i == 0)
    def _():
        out[...] = jnp.zeros((1, 1), jnp.float32)

    def score(hrow, hq, rrow, rq, trow, tq):
        hev, hod = _unpack(hrow[...], hq[...])
        rev, rod = _unpack(rrow[...], rq[...])
        tev, tod = _unpack(trow[...], tq[...])
        z = jnp.dot(hev, w1ae[...], preferred_element_type=jnp.float32)
        z += jnp.dot(hod, w1ao[...], preferred_element_type=jnp.float32)
        z += jnp.dot(rev, w1be[...], preferred_element_type=jnp.float32)
        z += jnp.dot(rod, w1bo[...], preferred_element_type=jnp.float32)
        z = jnp.maximum(z + b1[...], 0.0)
        z = jnp.maximum(
            jnp.dot(z, w2[...], preferred_element_type=jnp.float32) + b2[...], 0.0)
        oe = jnp.dot(z, w3e[...], preferred_element_type=jnp.float32) + b3e[...]
        oo = jnp.dot(z, w3o[...], preferred_element_type=jnp.float32) + b3o[...]
        de = oe - tev
        do = oo - tod
        return jnp.sqrt(jnp.sum(de * de + do * do, axis=1))

    ps = score(ph[...], qph[...], pr[...], qpr[...], pt[...], qpt[...])
    ns = score(nh[...], qnh[...], nr[...], qnr[...], nt[...], qnt[...])
    part = jnp.sum(jnp.maximum(_MARGIN + ps - ns, 0.0))
    out[...] += jnp.full((1, 1), part * (1.0 / _BATCH), jnp.float32)


def _mlp_loss(ent_rows, rel_rows, ent_q, rel_q, W1, b1, W2, b2, W3, b3):
    B = _BATCH
    T = 1024
    nblk = B // T

    def row_spec(off):
        return pl.BlockSpec((T, 128), lambda i, o=off: (i + o, 0))

    def q_spec(off):
        return pl.BlockSpec((T, 1), lambda i, o=off: (i + o, 0))

    def full(shape):
        return pl.BlockSpec(shape, lambda i, s=shape: tuple(0 for _ in s))

    w1ae = W1[0:_DEPTH:2]
    w1ao = W1[1:_DEPTH:2]
    w1be = W1[_DEPTH::2]
    w1bo = W1[_DEPTH + 1::2]
    w3e = W3[:, 0::2]
    w3o = W3[:, 1::2]
    b3e = b3[0::2].reshape(1, _HALF)
    b3o = b3[1::2].reshape(1, _HALF)

    res = pl.pallas_call(
        _mlp_loss_body,
        grid=(nblk,),
        in_specs=[
            row_spec(0),
            row_spec(0),
            row_spec(nblk),
            row_spec(2 * nblk),
            row_spec(nblk),
            row_spec(3 * nblk),
            q_spec(0), q_spec(0), q_spec(nblk),
            q_spec(2 * nblk), q_spec(nblk), q_spec(3 * nblk),
            full((_HALF, 128)), full((_HALF, 128)),
            full((_HALF, 128)), full((_HALF, 128)),
            full((1, 128)),
            full((128, 128)),
            full((1, 128)),
            full((128, _HALF)), full((128, _HALF)),
            full((1, _HALF)), full((1, _HALF)),
        ],
        out_specs=pl.BlockSpec((1, 1), lambda i: (0, 0)),
        out_shape=jax.ShapeDtypeStruct((1, 1), jnp.float32),
        compiler_params=pltpu.CompilerParams(
            dimension_semantics=("arbitrary",)),
    )(
        ent_rows, rel_rows, ent_rows, ent_rows, rel_rows, ent_rows,
        ent_q, rel_q, ent_q, ent_q, rel_q, ent_q,
        w1ae, w1ao, w1be, w1bo, b1.reshape(1, 128), W2, b2.reshape(1, 128),
        w3e, w3o, b3e, b3o,
    )
    return res[0, 0]


def _pack_idx(idx):
    p = (idx // (4 * _W)) * _W + (idx % _W)
    q = ((idx // _W) % 4).astype(jnp.float32)
    return p, q


def kernel(pos_x, neg_x, ent_emb, rel_emb, W1, b1, W2, b2, W3, b3):
    B = _BATCH
    idx_ent = jnp.concatenate(
        [pos_x[:, 0], pos_x[:, 1], neg_x[:, 0], neg_x[:, 1]])
    idx_rel = jnp.concatenate([pos_x[:, 2], neg_x[:, 2]])
    pe, qe = _pack_idx(idx_ent)
    pr, qr = _pack_idx(idx_rel)

    f = jnp.arange(_DEPTH)
    m = jnp.arange(_HALF)
    Eev = (f[:, None] == 2 * m[None, :]).astype(jnp.bfloat16)
    Eod = (f[:, None] == 2 * m[None, :] + 1).astype(jnp.bfloat16)
    Z = jnp.zeros((_DEPTH, _HALF), jnp.bfloat16)
    eel = jnp.concatenate([Eev, Z], axis=1)
    eeh = jnp.concatenate([Z, Eev], axis=1)
    eol = jnp.concatenate([Eod, Z], axis=1)
    eoh = jnp.concatenate([Z, Eod], axis=1)

    ent_pack = _tc_pack_table(ent_emb, eel, eeh, eol, eoh)
    rel_pack = _tc_pack_table(rel_emb, eel, eeh, eol, eoh)
    ent_rows = _sc_gather(ent_pack, pe)
    rel_rows = _sc_gather(rel_pack, pr)
    return _mlp_loss(ent_rows, rel_rows, qe.reshape(4 * B, 1),
                     qr.reshape(2 * B, 1), W1, b1, W2, b2, W3, b3)


# speedup vs baseline: 1.1727x; 1.1727x over previous
import functools

import jax
import jax.numpy as jnp
from jax import lax
from jax.experimental import pallas as pl
from jax.experimental.pallas import tpu as pltpu
from jax.experimental.pallas import tpu_sc as plsc

_BATCH = 16384
_DEPTH = 64
_HALF = 32
_MARGIN = 1.0

_NC = 2
_NS = 16
_NW = _NC * _NS
_CHUNK = 128
_NBUF = 4

_W = 8192


def _pack_body(x, eel, eeh, eol, eoh, out):
    xb = x[...].astype(jnp.bfloat16)
    c0 = xb[:, 0:_W]
    c1 = xb[:, _W:2 * _W]
    c2 = xb[:, 2 * _W:3 * _W]
    c3 = xb[:, 3 * _W:4 * _W]

    def dots(a, b, el, eh):
        z = jax.lax.dot_general(a, el[...], (((0,), (0,)), ((), ())),
                                preferred_element_type=jnp.float32)
        z += jax.lax.dot_general(b, eh[...], (((0,), (0,)), ((), ())),
                                 preferred_element_type=jnp.float32)
        return jax.lax.bitcast_convert_type(z, jnp.uint32)

    def pack(a, b):
        ue = dots(a, b, eel, eeh)
        uo = dots(a, b, eol, eoh)
        return jax.lax.bitcast_convert_type(
            (ue >> 16) | (uo & jnp.uint32(0xFFFF0000)), jnp.float32)

    out[...] = jnp.concatenate([pack(c0, c1), pack(c2, c3)], axis=1)


def _tc_pack_table(table, eel, eeh, eol, eoh):
    at = table.T
    n = at.shape[1]
    nin = -(-n // _W)
    nsb = -(-n // (4 * _W))

    def e_spec():
        return pl.BlockSpec((_DEPTH, _DEPTH), lambda i: (0, 0))

    return pl.pallas_call(
        _pack_body,
        grid=(nsb,),
        in_specs=[pl.BlockSpec((_DEPTH, 4 * _W), lambda i: (0, i)),
                  e_spec(), e_spec(), e_spec(), e_spec()],
        out_specs=pl.BlockSpec((_W, 2 * _DEPTH), lambda i: (i, 0)),
        out_shape=jax.ShapeDtypeStruct((nsb * _W, 2 * _DEPTH), jnp.float32),
        compiler_params=pltpu.CompilerParams(
            dimension_semantics=("arbitrary",)),
    )(at, eel, eeh, eol, eoh)


def _sc_gather(pairs, idx):
    n = idx.shape[0]
    per_w = n // _NW
    groups = per_w // (_CHUNK * _NBUF)
    mesh = plsc.VectorSubcoreMesh(core_axis_name="c", subcore_axis_name="s")

    @functools.partial(
        pl.kernel,
        mesh=mesh,
        out_type=jax.ShapeDtypeStruct((n, 128), jnp.float32),
        scratch_types=[
            [pltpu.VMEM((_CHUNK,), jnp.int32) for _ in range(_NBUF)],
            [pltpu.VMEM((_CHUNK, 128), jnp.float32) for _ in range(_NBUF)],
            [pltpu.SemaphoreType.DMA for _ in range(_NBUF)],
        ],
    )
    def gather_k(tab_hbm, idx_hbm, out_hbm, idx_vs, rows_vs, sems):
        wid = lax.axis_index("s") * _NC + lax.axis_index("c")

        def body(g, carry):
            base = wid * per_w + g * (_CHUNK * _NBUF)
            copies = []
            for b in range(_NBUF):
                off = base + b * _CHUNK
                pltpu.sync_copy(idx_hbm.at[pl.ds(off, _CHUNK)], idx_vs[b])
                copies.append(
                    pltpu.async_copy(tab_hbm.at[idx_vs[b]], rows_vs[b], sems[b]))
            for b in range(_NBUF):
                off = base + b * _CHUNK
                copies[b].wait()
                pltpu.sync_copy(rows_vs[b], out_hbm.at[pl.ds(off, _CHUNK)])
            return carry

        lax.fori_loop(0, groups, body, 0)

    return gather_k(pairs, idx)


def _unpack(rows, q):
    w = (rows[:, 0:_HALF] * (q == 0.0).astype(jnp.float32)
         + rows[:, _HALF:2 * _HALF] * (q == 1.0).astype(jnp.float32)
         + rows[:, 2 * _HALF:3 * _HALF] * (q == 2.0).astype(jnp.float32)
         + rows[:, 3 * _HALF:4 * _HALF] * (q == 3.0).astype(jnp.float32))
    u = jax.lax.bitcast_convert_type(w, jnp.uint32)
    ev = jax.lax.bitcast_convert_type(u << 16, jnp.float32)
    od = jax.lax.bitcast_convert_type(u & jnp.uint32(0xFFFF0000), jnp.float32)
    return ev, od


def _mlp_loss_body(ph, pr, pt, nh, nr, nt, qph, qpr, qpt, qnh, qnr, qnt,
                   w1ae, w1ao, w1be, w1bo, b1, w2, b2, w3e, w3o, b3e, b3o,
                   out):
    i = pl.program_id(0)

    @pl.when(i == 0)
    def _():
        out[...] = jnp.zeros((1, 1), jnp.float32)

    def score(hrow, hq, rrow, rq, trow, tq):
        hev, hod = _unpack(hrow[...], hq[...])
        rev, rod = _unpack(rrow[...], rq[...])
        tev, tod = _unpack(trow[...], tq[...])
        z = jnp.dot(hev, w1ae[...], preferred_element_type=jnp.float32)
        z += jnp.dot(hod, w1ao[...], preferred_element_type=jnp.float32)
        z += jnp.dot(rev, w1be[...], preferred_element_type=jnp.float32)
        z += jnp.dot(rod, w1bo[...], preferred_element_type=jnp.float32)
        z = jnp.maximum(z + b1[...], 0.0)
        z = jnp.maximum(
            jnp.dot(z, w2[...], preferred_element_type=jnp.float32) + b2[...], 0.0)
        oe = jnp.dot(z, w3e[...], preferred_element_type=jnp.float32) + b3e[...]
        oo = jnp.dot(z, w3o[...], preferred_element_type=jnp.float32) + b3o[...]
        de = oe - tev
        do = oo - tod
        return jnp.sqrt(jnp.sum(de * de + do * do, axis=1))

    ps = score(ph[...], qph[...], pr[...], qpr[...], pt[...], qpt[...])
    ns = score(nh[...], qnh[...], nr[...], qnr[...], nt[...], qnt[...])
    part = jnp.sum(jnp.maximum(_MARGIN + ps - ns, 0.0))
    out[...] += jnp.full((1, 1), part * (1.0 / _BATCH), jnp.float32)


def _mlp_loss(ent_rows, rel_rows, ent_q, rel_q, W1, b1, W2, b2, W3, b3):
    B = _BATCH
    T = 1024
    nblk = B // T

    def row_spec(off):
        return pl.BlockSpec((T, 128), lambda i, o=off: (i + o, 0))

    def q_spec(off):
        return pl.BlockSpec((T, 1), lambda i, o=off: (i + o, 0))

    def full(shape):
        return pl.BlockSpec(shape, lambda i, s=shape: tuple(0 for _ in s))

    w1ae = W1[0:_DEPTH:2]
    w1ao = W1[1:_DEPTH:2]
    w1be = W1[_DEPTH::2]
    w1bo = W1[_DEPTH + 1::2]
    w3e = W3[:, 0::2]
    w3o = W3[:, 1::2]
    b3e = b3[0::2].reshape(1, _HALF)
    b3o = b3[1::2].reshape(1, _HALF)

    res = pl.pallas_call(
        _mlp_loss_body,
        grid=(nblk,),
        in_specs=[
            row_spec(0),
            row_spec(0),
            row_spec(nblk),
            row_spec(2 * nblk),
            row_spec(nblk),
            row_spec(3 * nblk),
            q_spec(0), q_spec(0), q_spec(nblk),
            q_spec(2 * nblk), q_spec(nblk), q_spec(3 * nblk),
            full((_HALF, 128)), full((_HALF, 128)),
            full((_HALF, 128)), full((_HALF, 128)),
            full((1, 128)),
            full((128, 128)),
            full((1, 128)),
            full((128, _HALF)), full((128, _HALF)),
            full((1, _HALF)), full((1, _HALF)),
        ],
        out_specs=pl.BlockSpec((1, 1), lambda i: (0, 0)),
        out_shape=jax.ShapeDtypeStruct((1, 1), jnp.float32),
        compiler_params=pltpu.CompilerParams(
            dimension_semantics=("arbitrary",)),
    )(
        ent_rows, rel_rows, ent_rows, ent_rows, rel_rows, ent_rows,
        ent_q, rel_q, ent_q, ent_q, rel_q, ent_q,
        w1ae, w1ao, w1be, w1bo, b1.reshape(1, 128), W2, b2.reshape(1, 128),
        w3e, w3o, b3e, b3o,
    )
    return res[0, 0]


def _pack_idx(idx):
    p = (idx // (4 * _W)) * _W + (idx % _W)
    q = ((idx // _W) % 4).astype(jnp.float32)
    return p, q


def kernel(pos_x, neg_x, ent_emb, rel_emb, W1, b1, W2, b2, W3, b3):
    B = _BATCH
    idx_ent = jnp.concatenate(
        [pos_x[:, 0], pos_x[:, 1], neg_x[:, 0], neg_x[:, 1]])
    idx_rel = jnp.concatenate([pos_x[:, 2], neg_x[:, 2]])
    pe, qe = _pack_idx(idx_ent)
    pr, qr = _pack_idx(idx_rel)

    f = jnp.arange(_DEPTH)
    m = jnp.arange(_HALF)
    Eev = (f[:, None] == 2 * m[None, :]).astype(jnp.bfloat16)
    Eod = (f[:, None] == 2 * m[None, :] + 1).astype(jnp.bfloat16)
    Z = jnp.zeros((_DEPTH, _HALF), jnp.bfloat16)
    eel = jnp.concatenate([Eev, Z], axis=1)
    eeh = jnp.concatenate([Z, Eev], axis=1)
    eol = jnp.concatenate([Eod, Z], axis=1)
    eoh = jnp.concatenate([Z, Eod], axis=1)

    ent_pack = _tc_pack_table(ent_emb, eel, eeh, eol, eoh)
    rel_pack = _tc_pack_table(rel_emb, eel, eeh, eol, eoh)
    ent_rows = _sc_gather(ent_pack, pe)
    rel_rows = _sc_gather(rel_pack, pr)
    return _mlp_loss(ent_rows, rel_rows, qe.reshape(4 * B, 1),
                     qr.reshape(2 * B, 1), W1, b1, W2, b2, W3, b3)
